# baseline (speedup 1.0000x reference)
import jax
import jax.numpy as jnp
from jax import lax
from jax.experimental import pallas as pl
from jax.experimental.pallas import tpu as pltpu
from jax.experimental.pallas import tpu_sc as plsc

J = 26
NI = 16384
NV = 1000000
D = 32
HD = 16
NS = 16
CW = 1536
NCHUNK = 651
NIT = 128

_mesh = plsc.VectorSubcoreMesh(core_axis_name="c", subcore_axis_name="s")


def _body(xf, tT, out5, srm, vin, vout, idxraw, idxbuf, g, tbuf,
          si0, si1, so0, so1, sg0, sg1, sg2, sx, st0, st1):
    c = lax.axis_index("c")
    s = lax.axis_index("s")
    base_d = HD * c

    lane = lax.iota(jnp.int32, 16)
    iota26 = lane * 26
    iota16 = lane * 16

    svin = (si0, si1)
    svout = (so0, so1)
    sgs = (sg0, sg1, sg2)
    sts = (st0, st1)

    def p1_c0(k):
        return pl.multiple_of(k * CW, 128)

    def p1_fire(k, buf):
        c0 = p1_c0(k)
        pltpu.async_copy(tT.at[pl.ds(base_d, HD), pl.ds(c0, CW)],
                         vin.at[buf], svin[buf])

    def p1_wait_in(buf):
        pltpu.make_async_copy(tT.at[pl.ds(base_d, HD), pl.ds(0, CW)],
                              vin.at[buf], svin[buf]).wait()

    def p1_transpose(buf):
        def rbody(rr, carry):
            r0 = rr * 8
            for u in range(8):
                r = r0 + u
                v = plsc.load_gather(vin.at[buf],
                                     [lane, jnp.zeros((16,), jnp.int32) + r])
                vout[buf, r, :] = v
            return carry
        lax.fori_loop(0, CW // 8, rbody, 0)

    def p1_out(k, buf):
        c0 = p1_c0(k)
        pltpu.async_copy(vout.at[buf], srm.at[c, pl.ds(c0, CW)], svout[buf])

    def p1_wait_out(buf):
        pltpu.make_async_copy(vout.at[buf], srm.at[c, pl.ds(0, CW)],
                              svout[buf]).wait()

    n1 = 41

    p1_fire(s, 0)

    def p1_body(i, carry):
        k = s + 16 * i
        buf = lax.rem(i, 2)
        knext = k + 16

        @pl.when(knext < NCHUNK)
        def _():
            p1_fire_dyn(knext, 1 - buf)

        @pl.when(k < NCHUNK)
        def _():
            p1_wait_in_dyn(buf)
            @pl.when(i >= 2)
            def _():
                p1_wait_out_dyn(buf)
            p1_transpose_dyn(buf)
            p1_out_dyn(k, buf)
        return carry

    def p1_wait_in_dyn(buf):
        @pl.when(buf == 0)
        def _():
            p1_wait_in(0)
        @pl.when(buf == 1)
        def _():
            p1_wait_in(1)

    def p1_fire_dyn(k, buf):
        @pl.when(buf == 0)
        def _():
            p1_fire(k, 0)
        @pl.when(buf == 1)
        def _():
            p1_fire(k, 1)

    def p1_transpose_dyn(buf):
        @pl.when(buf == 0)
        def _():
            p1_transpose(0)
        @pl.when(buf == 1)
        def _():
            p1_transpose(1)

    def p1_out_dyn(k, buf):
        @pl.when(buf == 0)
        def _():
            p1_out(k, 0)
        @pl.when(buf == 1)
        def _():
            p1_out(k, 1)

    def p1_wait_out_dyn(buf):
        @pl.when(buf == 0)
        def _():
            p1_wait_out(0)
        @pl.when(buf == 1)
        def _():
            p1_wait_out(1)

    lax.fori_loop(0, n1, p1_body, 0)
    p1_wait_out(0)
    p1_wait_out(1)

    @pl.when(s == 0)
    def _():
        pltpu.sync_copy(tT.at[pl.ds(base_d, HD), pl.ds(999936, 64)],
                        vin.at[0, :, pl.ds(0, 64)])

        def tail_body(rr, carry):
            r0 = rr * 8
            for u in range(8):
                r = r0 + u
                v = plsc.load_gather(vin.at[0],
                                     [lane, jnp.zeros((16,), jnp.int32) + r])
                vout[0, r, :] = v
            return carry
        lax.fori_loop(0, 8, tail_body, 0)
        pltpu.sync_copy(vout.at[0, pl.ds(0, 64)],
                        srm.at[c, pl.ds(999936, 64)])

    plsc.subcore_barrier()

    def p2_gather(j, gbuf):
        pltpu.async_copy(srm.at[c].at[idxbuf.at[j]], g.at[gbuf], sgs[gbuf])

    def p2_wait_gather(gbuf):
        pltpu.make_async_copy(srm.at[c, pl.ds(0, NIT)], g.at[gbuf],
                              sgs[gbuf]).wait()

    def p2_transpose(gbuf, tb):
        def kbody(dd, carry):
            for u in range(8):
                v = plsc.load_gather(
                    g.at[gbuf],
                    [u * 16 + lane, jnp.zeros((16,), jnp.int32) + dd])
                tbuf[tb, dd, pl.ds(u * 16, 16)] = v
            return carry
        lax.fori_loop(0, HD, kbody, 0)

    def p2_out(j, it, tb):
        pltpu.async_copy(tbuf.at[tb, pl.ds(0, 8)],
                         out5.at[j, 2 * c, it], sts[tb])
        pltpu.async_copy(tbuf.at[tb, pl.ds(8, 8)],
                         out5.at[j, 2 * c + 1, it], sts[tb])

    def p2_wait_out(tb):
        pltpu.make_async_copy(tbuf.at[tb], out5.at[0, pl.ds(0, 2), 0],
                              sts[tb]).wait()

    def p2_gather_dyn(j, gbuf):
        for b in range(3):
            @pl.when(gbuf == b)
            def _(b=b):
                p2_gather(j, b)

    def p2_wait_gather_dyn(gbuf):
        for b in range(3):
            @pl.when(gbuf == b)
            def _(b=b):
                p2_wait_gather(b)

    def p2_transpose_dyn(gbuf, tb):
        for b in range(3):
            for t in range(2):
                @pl.when((gbuf == b) & (tb == t))
                def _(b=b, t=t):
                    p2_transpose(b, t)

    def p2_out_dyn(j, it, tb):
        for t in range(2):
            @pl.when(tb == t)
            def _(t=t):
                p2_out(j, it, t)

    def p2_wait_out_dyn(tb):
        for t in range(2):
            @pl.when(tb == t)
            def _(t=t):
                p2_wait_out(t)

    def it_body(it8, carry):
        it = s + 16 * it8
        pltpu.async_copy(xf.at[pl.ds(it * 3328, 3328)], idxraw, sx)
        pltpu.make_async_copy(xf.at[pl.ds(0, 3328)], idxraw, sx).wait()

        def dj(j, carry):
            for lg in range(8):
                v = plsc.load_gather(idxraw, [iota26 + (lg * 416 + j)])
                idxbuf[j, pl.ds(lg * 16, 16)] = v
            return carry
        lax.fori_loop(0, J, dj, 0)

        p2_gather_dyn(0, 0)
        p2_gather_dyn(1, 1)

        def jbody(j, carry):
            gbuf = lax.rem(j, 3)
            tb = lax.rem(j, 2)

            @pl.when(j + 2 < J)
            def _():
                p2_gather_dyn(j + 2, lax.rem(j + 2, 3))

            p2_wait_gather_dyn(gbuf)

            @pl.when(j >= 2)
            def _():
                p2_wait_out_dyn(tb)

            p2_transpose_dyn(gbuf, tb)
            p2_out_dyn(j, it, tb)
            return carry

        lax.fori_loop(0, J, jbody, 0)
        p2_wait_out(0)
        p2_wait_out(1)
        return carry

    lax.fori_loop(0, 8, it_body, 0)


_gather = pl.kernel(
    _body,
    out_type=jax.ShapeDtypeStruct((J, 4, NIT, 8, 128), jnp.float32),
    mesh=_mesh,
    scratch_types=[
        pltpu.HBM((2, NV, HD), jnp.float32),
        pltpu.VMEM((2, HD, CW), jnp.float32),
        pltpu.VMEM((2, CW, HD), jnp.float32),
        pltpu.VMEM((3328,), jnp.int32),
        pltpu.VMEM((J, NIT), jnp.int32),
        pltpu.VMEM((3, NIT, HD), jnp.float32),
        pltpu.VMEM((2, HD, 128), jnp.float32),
    ] + [pltpu.SemaphoreType.DMA] * 10,
    compiler_params=pltpu.CompilerParams(use_tc_tiling_on_sc=False,
                                         needs_layout_passes=False),
)


def kernel(x, table):
    xf = x.reshape(-1)
    tT = table.T
    out5 = _gather(xf, tT)
    return (out5.transpose(0, 1, 3, 2, 4)
                .reshape(J, D, NI)
                .transpose(2, 0, 1))


# speedup vs baseline: 4.2028x; 4.2028x over previous
import jax
import jax.numpy as jnp
from jax import lax
from jax.experimental import pallas as pl
from jax.experimental.pallas import tpu as pltpu
from jax.experimental.pallas import tpu_sc as plsc

J = 26
NI = 16384
NV = 1000000
D = 32
NIT = 128
ITS_PER_W = 4

_mesh = plsc.VectorSubcoreMesh(core_axis_name="c", subcore_axis_name="s")


def _body(xf, table, out5, idxraw, idxbuf, g, tbuf,
          sx, sg0, sg1, sg2, st0, st1):
    c = lax.axis_index("c")
    s = lax.axis_index("s")
    w = s * 2 + c

    lane = lax.iota(jnp.int32, 16)
    iota26 = lane * 26

    sgs = (sg0, sg1, sg2)
    sts = (st0, st1)

    def gather(j, gbuf):
        pltpu.async_copy(table.at[idxbuf.at[j]], g.at[gbuf], sgs[gbuf])

    def wait_gather(gbuf):
        pltpu.make_async_copy(table.at[pl.ds(0, NIT)], g.at[gbuf],
                              sgs[gbuf]).wait()

    def transpose(gbuf, tb):
        def kbody(dd, carry):
            dt = dd // 8
            ds_ = lax.rem(dd, 8)
            for u in range(8):
                v = plsc.load_gather(
                    g.at[gbuf],
                    [u * 16 + lane, jnp.zeros((16,), jnp.int32) + dd])
                tbuf[tb, dt, ds_, pl.ds(u * 16, 16)] = v
            return carry
        lax.fori_loop(0, D, kbody, 0)

    def out(j, it, tb):
        pltpu.async_copy(tbuf.at[tb], out5.at[j, pl.ds(0, 4), it], sts[tb])

    def wait_out(tb):
        pltpu.make_async_copy(tbuf.at[tb], out5.at[0, pl.ds(0, 4), 0],
                              sts[tb]).wait()

    def gather_dyn(j, gbuf):
        for b in range(3):
            @pl.when(gbuf == b)
            def _(b=b):
                gather(j, b)

    def wait_gather_dyn(gbuf):
        for b in range(3):
            @pl.when(gbuf == b)
            def _(b=b):
                wait_gather(b)

    def transpose_dyn(gbuf, tb):
        for b in range(3):
            for t in range(2):
                @pl.when((gbuf == b) & (tb == t))
                def _(b=b, t=t):
                    transpose(b, t)

    def out_dyn(j, it, tb):
        for t in range(2):
            @pl.when(tb == t)
            def _(t=t):
                out(j, it, t)

    def wait_out_dyn(tb):
        for t in range(2):
            @pl.when(tb == t)
            def _(t=t):
                wait_out(t)

    def it_body(it8, carry):
        it = w * ITS_PER_W + it8
        pltpu.async_copy(xf.at[pl.ds(it * (NIT * J), NIT * J)], idxraw, sx)
        pltpu.make_async_copy(xf.at[pl.ds(0, NIT * J)], idxraw, sx).wait()

        def dj(j, carry):
            for lg in range(8):
                v = plsc.load_gather(idxraw, [iota26 + (lg * 16 * J + j)])
                idxbuf[j, pl.ds(lg * 16, 16)] = v
            return carry
        lax.fori_loop(0, J, dj, 0)

        gather_dyn(0, 0)
        gather_dyn(1, 1)

        def jbody(j, carry):
            gbuf = lax.rem(j, 3)
            tb = lax.rem(j, 2)

            @pl.when(j + 2 < J)
            def _():
                gather_dyn(j + 2, lax.rem(j + 2, 3))

            wait_gather_dyn(gbuf)

            @pl.when(j >= 2)
            def _():
                wait_out_dyn(tb)

            transpose_dyn(gbuf, tb)
            out_dyn(j, it, tb)
            return carry

        lax.fori_loop(0, J, jbody, 0)
        wait_out(0)
        wait_out(1)
        return carry

    lax.fori_loop(0, ITS_PER_W, it_body, 0)


_gather = pl.kernel(
    _body,
    out_type=jax.ShapeDtypeStruct((J, 4, NIT, 8, 128), jnp.float32),
    mesh=_mesh,
    scratch_types=[
        pltpu.VMEM((NIT * J,), jnp.int32),
        pltpu.VMEM((J, NIT), jnp.int32),
        pltpu.VMEM((3, NIT, D), jnp.float32),
        pltpu.VMEM((2, 4, 8, 128), jnp.float32),
    ] + [pltpu.SemaphoreType.DMA] * 6,
    compiler_params=pltpu.CompilerParams(use_tc_tiling_on_sc=False,
                                         needs_layout_passes=False),
)


def kernel(x, table):
    xf = x.reshape(-1)
    out5 = _gather(xf, table)
    return (out5.transpose(0, 1, 3, 2, 4)
                .reshape(J, D, NI)
                .transpose(2, 0, 1))
